# baseline (speedup 1.0000x reference)
import jax
import jax.numpy as jnp
from jax.experimental import pallas as pl
from jax.experimental.pallas import tpu as pltpu

_BBLK = 64


def _body(t0_ref, t1_ref, i0_ref, rem_ref, ridx_ref, mtok_ref, pe_ref,
          o0_ref, o1_ref, o2_ref, orv_ref):
    o0_ref[...] = t0_ref[...] + pe_ref[1, :]
    o1_ref[...] = t1_ref[...] + pe_ref[2, :]
    o2_ref[...] = i0_ref[...] + pe_ref[3, :]
    rem0 = rem_ref[:, 0, :]
    rem1 = rem_ref[:, 1, :]
    mask = mtok_ref[0, :]
    idx = ridx_ref[...]
    orv_ref[:, 0, :] = rem0 + pe_ref[4, :]
    for j in range(3):
        sel = idx[:, j:j + 1] == 0
        row = jnp.where(sel, rem1, mask[None, :])
        if j < 2:
            row = row + pe_ref[5 + j, :]
        orv_ref[:, j + 1, :] = row


def kernel(temporal_t0, temporal_t1, img_i0, others_remain_data, mask_token,
           revert_idx, pos_emb):
    b, t, d = temporal_t0.shape
    bblk = _BBLK
    grid = (b // bblk,)
    t0r = temporal_t0.reshape(b * t, d)
    t1r = temporal_t1.reshape(b * t, d)
    i0r = img_i0.reshape(b * t, d)

    big_in = pl.BlockSpec((bblk * t, d), lambda i: (i, 0))
    rem_in = pl.BlockSpec((bblk, 2, d), lambda i: (i, 0, 0))
    ridx_in = pl.BlockSpec((bblk, 3), lambda i: (i, 0))
    mtok_in = pl.BlockSpec((1, d), lambda i: (0, 0))
    pe_in = pl.BlockSpec((7, d), lambda i: (0, 0))
    orv_out = pl.BlockSpec((bblk, 4, d), lambda i: (i, 0, 0))

    o0, o1, o2, orv = pl.pallas_call(
        _body,
        grid=grid,
        in_specs=[big_in, big_in, big_in, rem_in, ridx_in, mtok_in, pe_in],
        out_specs=[big_in, big_in, big_in, orv_out],
        out_shape=[
            jax.ShapeDtypeStruct((b * t, d), jnp.float32),
            jax.ShapeDtypeStruct((b * t, d), jnp.float32),
            jax.ShapeDtypeStruct((b * t, d), jnp.float32),
            jax.ShapeDtypeStruct((b, 4, d), jnp.float32),
        ],
    )(t0r, t1r, i0r, others_remain_data, revert_idx, mask_token, pos_emb)
    return (o0.reshape(b, t, d), o1.reshape(b, t, d), o2.reshape(b, t, d), orv)


# speedup vs baseline: 1.8981x; 1.8981x over previous
import jax
import jax.numpy as jnp
from jax.experimental import pallas as pl
from jax.experimental.pallas import tpu as pltpu

_BBLK = 128


def _body(t0_ref, t1_ref, i0_ref, rem_ref, ridx_ref, mtok_ref, pe_ref,
          o0_ref, o1_ref, o2_ref, orv_ref):
    pe1 = pe_ref[1, :]
    pe2 = pe_ref[2, :]
    pe3 = pe_ref[3, :]
    o0_ref[...] = t0_ref[...] + pe1[None, None, :]
    o1_ref[...] = t1_ref[...] + pe2[None, None, :]
    o2_ref[...] = i0_ref[...] + pe3[None, None, :]
    rem0 = rem_ref[:, 0, :]
    rem1 = rem_ref[:, 1, :]
    mask = mtok_ref[0, :]
    idx = ridx_ref[...]
    orv_ref[:, 0, :] = rem0 + pe_ref[4, :]
    for j in range(3):
        sel = idx[:, j:j + 1] == 0
        row = jnp.where(sel, rem1, mask[None, :])
        if j < 2:
            row = row + pe_ref[5 + j, :]
        orv_ref[:, j + 1, :] = row


def kernel(temporal_t0, temporal_t1, img_i0, others_remain_data, mask_token,
           revert_idx, pos_emb):
    b, t, d = temporal_t0.shape
    bblk = _BBLK
    grid = (b // bblk,)

    big_in = pl.BlockSpec((bblk, t, d), lambda i: (i, 0, 0))
    rem_in = pl.BlockSpec((bblk, 2, d), lambda i: (i, 0, 0))
    ridx_in = pl.BlockSpec((bblk, 3), lambda i: (i, 0))
    mtok_in = pl.BlockSpec((1, d), lambda i: (0, 0))
    pe_in = pl.BlockSpec((7, d), lambda i: (0, 0))
    orv_out = pl.BlockSpec((bblk, 4, d), lambda i: (i, 0, 0))

    big_shape = jax.ShapeDtypeStruct((b, t, d), jnp.float32)
    o0, o1, o2, orv = pl.pallas_call(
        _body,
        grid=grid,
        in_specs=[big_in, big_in, big_in, rem_in, ridx_in, mtok_in, pe_in],
        out_specs=[big_in, big_in, big_in, orv_out],
        out_shape=[
            big_shape, big_shape, big_shape,
            jax.ShapeDtypeStruct((b, 4, d), jnp.float32),
        ],
    )(temporal_t0, temporal_t1, img_i0, others_remain_data, revert_idx,
      mask_token, pos_emb)
    return (o0, o1, o2, orv)
